# Initial kernel scaffold; baseline (speedup 1.0000x reference)
import jax
import jax.numpy as jnp
from jax.experimental import pallas as pl


def kernel(x, W0, b0, Rs):
    raise NotImplementedError("write your pallas kernel here")


# speedup vs baseline: 3.7173x; 3.7173x over previous
import functools

import jax
import jax.numpy as jnp
from jax import lax
from jax.experimental import pallas as pl
from jax.experimental.pallas import tpu as pltpu
from jax.experimental.pallas import tpu_sc as plsc

K = 32
RADIUS = 0.15
NROT = 24
NOUT = 16
C = NROT * NOUT


def _knn_proj_body(xq_ref, xt_ref, wct_ref, idx_ref, p_ref, *, m, k, r2):
    b = pl.program_id(0)
    xq = xq_ref[0]
    xt = xt_ref[0]
    p_ref[0] = jnp.dot(xq, wct_ref[...], preferred_element_type=jnp.float32)
    x2 = jnp.sum(xq * xq, axis=1, keepdims=True)
    y2 = jnp.sum(xt * xt, axis=0, keepdims=True)
    d2 = x2 + y2 - 2.0 * jnp.dot(xq, xt, preferred_element_type=jnp.float32)
    d2 = jnp.maximum(d2, 0.0)
    iota = lax.broadcasted_iota(jnp.int32, d2.shape, 1)
    offset = b * m
    first = None
    cols = []
    for _ in range(k):
        mn = jnp.min(d2, axis=1, keepdims=True)
        am = jnp.min(jnp.where(d2 <= mn, iota, m), axis=1, keepdims=True)
        if first is None:
            first = am
        cols.append(jnp.where(mn <= r2, am, first) + offset)
        d2 = jnp.where(iota == am, jnp.float32(jnp.inf), d2)
    idx_ref[0] = jnp.concatenate(cols, axis=1)


def _knn_and_project(x_pad, xt_pad, wct):
    b, n, _ = x_pad.shape
    m = xt_pad.shape[2]
    qt = 256
    grid = (b, n // qt)
    return pl.pallas_call(
        functools.partial(_knn_proj_body, m=m, k=K, r2=RADIUS * RADIUS),
        grid=grid,
        in_specs=[
            pl.BlockSpec((1, qt, 8), lambda i, j: (i, j, 0)),
            pl.BlockSpec((1, 8, m), lambda i, j: (i, 0, 0)),
            pl.BlockSpec((8, C), lambda i, j: (0, 0)),
        ],
        out_specs=[
            pl.BlockSpec((1, qt, K), lambda i, j: (i, j, 0)),
            pl.BlockSpec((1, qt, C), lambda i, j: (i, j, 0)),
        ],
        out_shape=[
            jax.ShapeDtypeStruct((b, n, K), jnp.int32),
            jax.ShapeDtypeStruct((b, n, C), jnp.float32),
        ],
    )(x_pad, xt_pad, wct)


def _make_sc_gather_max(nq, qpw):
    info = plsc.get_sparse_core_info()
    nc = info.num_cores
    mesh = plsc.VectorSubcoreMesh(core_axis_name="c", subcore_axis_name="s")
    inv_r = 1.0 / RADIUS

    @functools.partial(
        pl.kernel,
        mesh=mesh,
        out_type=jax.ShapeDtypeStruct((nq, NOUT), jnp.float32),
        scratch_types=[
            pltpu.VMEM((qpw * K,), jnp.int32),
            pltpu.VMEM((qpw, C), jnp.float32),
            pltpu.VMEM((K, C), jnp.float32),
            pltpu.VMEM((qpw, NOUT), jnp.float32),
            pltpu.VMEM((NOUT,), jnp.float32),
            pltpu.SemaphoreType.DMA,
        ],
    )
    def sc_kernel(p_hbm, idx_hbm, b0_hbm, out_hbm,
                  idx_v, pq_v, rows_v, out_v, b0_v, sem):
        wid = lax.axis_index("s") * nc + lax.axis_index("c")
        base = wid * qpw
        pltpu.sync_copy(idx_hbm.at[pl.ds(base * K, qpw * K)], idx_v)
        pltpu.sync_copy(p_hbm.at[pl.ds(base, qpw)], pq_v)
        pltpu.sync_copy(b0_hbm, b0_v)
        b0v = b0_v[...]

        def body(q, carry):
            pltpu.async_copy(
                p_hbm.at[idx_v.at[pl.ds(q * K, K)]], rows_v, sem).wait()
            acc = jnp.zeros((16,), jnp.float32)
            for i in range(NROT):
                g = rows_v[0, pl.ds(i * 16, 16)]
                for r in range(1, K):
                    g = jnp.maximum(g, rows_v[r, pl.ds(i * 16, 16)])
                pq = pq_v[q, pl.ds(i * 16, 16)]
                acc = acc + jnp.maximum((g - pq) * inv_r + b0v, 0.0)
            out_v[q, :] = acc * (1.0 / NROT)
            return carry

        lax.fori_loop(0, qpw, body, 0)
        pltpu.sync_copy(out_v, out_hbm.at[pl.ds(base, qpw)])

    return sc_kernel


def kernel(x, W0, b0, Rs):
    b, n, _ = x.shape
    d = jnp.einsum('ijj->ij', Rs)
    wc = d[:, None, :] * W0[None, :, :]
    wct = jnp.swapaxes(wc.reshape(C, 3), 0, 1)
    wct = jnp.pad(wct, ((0, 5), (0, 0)))
    x_pad = jnp.pad(x, ((0, 0), (0, 0), (0, 5)))
    xt_pad = jnp.pad(jnp.swapaxes(x, 1, 2), ((0, 0), (0, 5), (0, 0)))

    idx, p = _knn_and_project(x_pad, xt_pad, wct)
    nq = b * n
    nw = 32
    sc = _make_sc_gather_max(nq, nq // nw)
    out = sc(p.reshape(nq, C), idx.reshape(nq * K), b0)
    return jnp.swapaxes(out.reshape(b, n, NOUT), 1, 2)


# speedup vs baseline: 4.3642x; 1.1740x over previous
import functools

import jax
import jax.numpy as jnp
from jax import lax
from jax.experimental import pallas as pl
from jax.experimental.pallas import tpu as pltpu
from jax.experimental.pallas import tpu_sc as plsc

K = 32
RADIUS = 0.15
NROT = 24
NOUT = 16
C = NROT * NOUT


def _knn_proj_body(xq_ref, xt_ref, wct_ref, idx_ref, p_ref, *, m, k, r2):
    b = pl.program_id(0)
    xq = xq_ref[0]
    xt = xt_ref[0]
    p_ref[0] = jnp.dot(xq, wct_ref[...], preferred_element_type=jnp.float32)
    x2 = jnp.sum(xq * xq, axis=1, keepdims=True)
    y2 = jnp.sum(xt * xt, axis=0, keepdims=True)
    d2 = x2 + y2 - 2.0 * jnp.dot(xq, xt, preferred_element_type=jnp.float32)
    d2 = jnp.maximum(d2, 0.0)
    iota = lax.broadcasted_iota(jnp.int32, d2.shape, 1)
    offset = b * m
    first = None
    cols = []
    for _ in range(k):
        mn = jnp.min(d2, axis=1, keepdims=True)
        am = jnp.min(jnp.where(d2 <= mn, iota, m), axis=1, keepdims=True)
        if first is None:
            first = am
        cols.append(jnp.where(mn <= r2, am, first) + offset)
        d2 = jnp.where(iota == am, jnp.float32(jnp.inf), d2)
    idx_ref[0] = jnp.concatenate(cols, axis=1)


def _knn_and_project(x_pad, xt_pad, wct):
    b, n, _ = x_pad.shape
    m = xt_pad.shape[2]
    qt = 256
    grid = (b, n // qt)
    return pl.pallas_call(
        functools.partial(_knn_proj_body, m=m, k=K, r2=RADIUS * RADIUS),
        grid=grid,
        in_specs=[
            pl.BlockSpec((1, qt, 8), lambda i, j: (i, j, 0)),
            pl.BlockSpec((1, 8, m), lambda i, j: (i, 0, 0)),
            pl.BlockSpec((8, C), lambda i, j: (0, 0)),
        ],
        out_specs=[
            pl.BlockSpec((1, qt, K), lambda i, j: (i, j, 0)),
            pl.BlockSpec((1, qt, C), lambda i, j: (i, j, 0)),
        ],
        out_shape=[
            jax.ShapeDtypeStruct((b, n, K), jnp.int32),
            jax.ShapeDtypeStruct((b, n, C), jnp.float32),
        ],
    )(x_pad, xt_pad, wct)


def _make_sc_gather_max(nq, qpw):
    info = plsc.get_sparse_core_info()
    nc = info.num_cores
    mesh = plsc.VectorSubcoreMesh(core_axis_name="c", subcore_axis_name="s")
    inv_r = 1.0 / RADIUS

    nbuf = 2

    @functools.partial(
        pl.kernel,
        mesh=mesh,
        out_type=jax.ShapeDtypeStruct((nq, NOUT), jnp.float32),
        scratch_types=[
            pltpu.VMEM((qpw * K,), jnp.int32),
            pltpu.VMEM((qpw, C), jnp.float32),
            pltpu.VMEM((nbuf, K, C), jnp.float32),
            pltpu.VMEM((qpw, NOUT), jnp.float32),
            pltpu.VMEM((NOUT,), jnp.float32),
        ] + [pltpu.SemaphoreType.DMA] * nbuf,
    )
    def sc_kernel(p_hbm, idx_hbm, b0_hbm, out_hbm,
                  idx_v, pq_v, rows_v, out_v, b0_v, *sems):
        wid = lax.axis_index("s") * nc + lax.axis_index("c")
        base = wid * qpw
        pltpu.sync_copy(idx_hbm.at[pl.ds(base * K, qpw * K)], idx_v)
        pltpu.sync_copy(p_hbm.at[pl.ds(base, qpw)], pq_v)
        pltpu.sync_copy(b0_hbm, b0_v)
        b0v = b0_v[...]

        def dma(q, j):
            return pltpu.make_async_copy(
                p_hbm.at[idx_v.at[pl.ds(q * K, K)]], rows_v.at[j], sems[j])

        def compute(q, j):
            acc = jnp.zeros((16,), jnp.float32)
            for i in range(NROT):
                g = rows_v[j, 0, pl.ds(i * 16, 16)]
                for r in range(1, K):
                    g = jnp.maximum(g, rows_v[j, r, pl.ds(i * 16, 16)])
                pq = pq_v[q, pl.ds(i * 16, 16)]
                acc = acc + jnp.maximum((g - pq) * inv_r + b0v, 0.0)
            out_v[q, :] = acc * (1.0 / NROT)

        for j in range(nbuf - 1):
            dma(j, j).start()

        def group(g8, carry):
            q0 = g8 * nbuf
            for j in range(nbuf):
                q = q0 + j
                nxt = q + nbuf - 1

                @pl.when(nxt < qpw)
                def _():
                    dma(nxt, (j + nbuf - 1) % nbuf).start()

                dma(q, j).wait()
                compute(q, j)
            return carry

        lax.fori_loop(0, qpw // nbuf, group, 0)
        pltpu.sync_copy(out_v, out_hbm.at[pl.ds(base, qpw)])

    return sc_kernel


def kernel(x, W0, b0, Rs):
    b, n, _ = x.shape
    d = jnp.einsum('ijj->ij', Rs)
    wc = d[:, None, :] * W0[None, :, :]
    wct = jnp.swapaxes(wc.reshape(C, 3), 0, 1)
    wct = jnp.pad(wct, ((0, 5), (0, 0)))
    x_pad = jnp.pad(x, ((0, 0), (0, 0), (0, 5)))
    xt_pad = jnp.pad(jnp.swapaxes(x, 1, 2), ((0, 0), (0, 5), (0, 0)))

    idx, p = _knn_and_project(x_pad, xt_pad, wct)
    nq = b * n
    nw = 32
    sc = _make_sc_gather_max(nq, nq // nw)
    out = sc(p.reshape(nq, C), idx.reshape(nq * K), b0)
    return jnp.swapaxes(out.reshape(b, n, NOUT), 1, 2)


# speedup vs baseline: 7.4857x; 1.7152x over previous
import functools

import jax
import jax.numpy as jnp
from jax import lax
from jax.experimental import pallas as pl
from jax.experimental.pallas import tpu as pltpu
from jax.experimental.pallas import tpu_sc as plsc

K = 32
RADIUS = 0.15
NROT = 24
NOUT = 16
NGRP = 10
GRP_W = (1.0,) * 4 + (2.0,) * 6
NZERO = 8.0
C = NGRP * NOUT


def _knn_proj_body(xq_ref, xt_ref, wct_ref, idx_ref, p_ref, *, m, k, r2):
    b = pl.program_id(0)
    xq = xq_ref[0]
    xt = xt_ref[0]
    p_ref[0] = jnp.dot(xq, wct_ref[...], preferred_element_type=jnp.float32)
    x2 = jnp.sum(xq * xq, axis=1, keepdims=True)
    y2 = jnp.sum(xt * xt, axis=0, keepdims=True)
    d2 = x2 + y2 - 2.0 * jnp.dot(xq, xt, preferred_element_type=jnp.float32)
    d2 = jnp.maximum(d2, 0.0)
    iota = lax.broadcasted_iota(jnp.int32, d2.shape, 1)
    offset = b * m
    first = None
    cols = []
    for _ in range(k):
        mn = jnp.min(d2, axis=1, keepdims=True)
        am = jnp.min(jnp.where(d2 <= mn, iota, m), axis=1, keepdims=True)
        if first is None:
            first = am
        cols.append(jnp.where(mn <= r2, am, first) + offset)
        d2 = jnp.where(iota == am, jnp.float32(jnp.inf), d2)
    idx_ref[0] = jnp.concatenate(cols, axis=1)


def _knn_and_project(x_pad, xt_pad, wct):
    b, n, _ = x_pad.shape
    m = xt_pad.shape[2]
    qt = 256
    grid = (b, n // qt)
    return pl.pallas_call(
        functools.partial(_knn_proj_body, m=m, k=K, r2=RADIUS * RADIUS),
        grid=grid,
        in_specs=[
            pl.BlockSpec((1, qt, 8), lambda i, j: (i, j, 0)),
            pl.BlockSpec((1, 8, m), lambda i, j: (i, 0, 0)),
            pl.BlockSpec((8, C), lambda i, j: (0, 0)),
        ],
        out_specs=[
            pl.BlockSpec((1, qt, K), lambda i, j: (i, j, 0)),
            pl.BlockSpec((1, qt, C), lambda i, j: (i, j, 0)),
        ],
        out_shape=[
            jax.ShapeDtypeStruct((b, n, K), jnp.int32),
            jax.ShapeDtypeStruct((b, n, C), jnp.float32),
        ],
    )(x_pad, xt_pad, wct)


def _make_sc_gather_max(nq, qpw):
    info = plsc.get_sparse_core_info()
    nc = info.num_cores
    mesh = plsc.VectorSubcoreMesh(core_axis_name="c", subcore_axis_name="s")
    inv_r = 1.0 / RADIUS

    nbuf = 4

    @functools.partial(
        pl.kernel,
        mesh=mesh,
        compiler_params=pltpu.CompilerParams(use_tc_tiling_on_sc=False),
        out_type=jax.ShapeDtypeStruct((nq, NOUT), jnp.float32),
        scratch_types=[
            pltpu.VMEM((qpw * K,), jnp.int32),
            pltpu.VMEM((qpw, C), jnp.float32),
            pltpu.VMEM((nbuf, K, C), jnp.float32),
            pltpu.VMEM((qpw, NOUT), jnp.float32),
            pltpu.VMEM((NOUT,), jnp.float32),
        ] + [pltpu.SemaphoreType.DMA] * nbuf,
    )
    def sc_kernel(p_hbm, idx_hbm, b0_hbm, out_hbm,
                  idx_v, pq_v, rows_v, out_v, b0_v, *sems):
        wid = lax.axis_index("s") * nc + lax.axis_index("c")
        base = wid * qpw
        pltpu.sync_copy(idx_hbm.at[pl.ds(base * K, qpw * K)], idx_v)
        pltpu.sync_copy(p_hbm.at[pl.ds(base, qpw)], pq_v)
        pltpu.sync_copy(b0_hbm, b0_v)
        b0v = b0_v[...]

        def dma(q, j):
            return pltpu.make_async_copy(
                p_hbm.at[idx_v.at[pl.ds(q * K, K)]], rows_v.at[j], sems[j])

        def compute(q, j):
            acc = NZERO * jnp.maximum(b0v, 0.0)
            for i in range(NGRP):
                g = rows_v[j, 0, pl.ds(i * 16, 16)]
                for r in range(1, K):
                    g = jnp.maximum(g, rows_v[j, r, pl.ds(i * 16, 16)])
                pq = pq_v[q, pl.ds(i * 16, 16)]
                t = jnp.maximum((g - pq) * inv_r + b0v, 0.0)
                acc = acc + (t + t if GRP_W[i] == 2.0 else t)
            out_v[q, :] = acc * (1.0 / NROT)

        for j in range(nbuf - 1):
            dma(j, j).start()

        def group(g8, carry):
            q0 = g8 * nbuf
            for j in range(nbuf):
                q = q0 + j
                nxt = q + nbuf - 1

                @pl.when(nxt < qpw)
                def _():
                    dma(nxt, (j + nbuf - 1) % nbuf).start()

                dma(q, j).wait()
                compute(q, j)
            return carry

        lax.fori_loop(0, qpw // nbuf, group, 0)
        pltpu.sync_copy(out_v, out_hbm.at[pl.ds(base, qpw)])

    return sc_kernel


def kernel(x, W0, b0, Rs):
    b, n, _ = x.shape
    d = jnp.einsum('ijj->ij', Rs)
    nnz = jnp.sum(jnp.abs(d) > 0.5, axis=1)
    d_sorted = d[jnp.argsort(-nnz, stable=True)]
    d_full = d_sorted[:4]
    d_ax12 = d_sorted[4:16]
    akey = jnp.argmax(jnp.abs(d_ax12), axis=1) * 2 + (
        jnp.sum(d_ax12, axis=1) > 0).astype(jnp.int32)
    d_axis = d_ax12[jnp.argsort(akey, stable=True)][::2]
    dg = jnp.concatenate([d_full, d_axis], axis=0)
    wc = dg[:, None, :] * W0[None, :, :]
    wct = jnp.swapaxes(wc.reshape(C, 3), 0, 1)
    wct = jnp.pad(wct, ((0, 5), (0, 0)))
    x_pad = jnp.pad(x, ((0, 0), (0, 0), (0, 5)))
    xt_pad = jnp.pad(jnp.swapaxes(x, 1, 2), ((0, 0), (0, 5), (0, 0)))

    idx, p = _knn_and_project(x_pad, xt_pad, wct)
    nq = b * n
    nw = 32
    sc = _make_sc_gather_max(nq, nq // nw)
    out = sc(p.reshape(nq, C), idx.reshape(nq * K), b0)
    return jnp.swapaxes(out.reshape(b, n, NOUT), 1, 2)


# speedup vs baseline: 9.4927x; 1.2681x over previous
import functools

import jax
import jax.numpy as jnp
import numpy as np
from jax import lax
from jax.experimental import pallas as pl
from jax.experimental.pallas import tpu as pltpu
from jax.experimental.pallas import tpu_sc as plsc

K = 32
RADIUS = 0.15
NROT = 24
NOUT = 16
NGRP = 10
GRP_W = (1.0,) * 4 + (2.0,) * 6
NZERO = 8.0
C = NGRP * NOUT


def _knn_proj_body(xq_ref, xt_ref, wct_ref, idx_ref, p_ref, *, m, k, r2):
    b = pl.program_id(0)
    xq = xq_ref[0]
    xt = xt_ref[0]
    p_ref[0] = jnp.dot(xq, wct_ref[...], preferred_element_type=jnp.float32)
    x2 = jnp.sum(xq * xq, axis=1, keepdims=True)
    y2 = jnp.sum(xt * xt, axis=0, keepdims=True)
    d2 = x2 + y2 - 2.0 * jnp.dot(xq, xt, preferred_element_type=jnp.float32)
    d2 = jnp.maximum(d2, 0.0)
    iota = lax.broadcasted_iota(jnp.int32, d2.shape, 1)
    key = (lax.bitcast_convert_type(d2, jnp.int32) & jnp.int32(~0x7FF)) | iota
    r2_bits = jnp.int32(np.float32(r2).view(np.int32) & ~0x7FF)
    offset = b * m
    first = None
    cols = []
    for _ in range(k):
        mk = jnp.min(key, axis=1, keepdims=True)
        am = mk & jnp.int32(0x7FF)
        if first is None:
            first = am
        cols.append(jnp.where(mk - am <= r2_bits, am, first) + offset)
        key = jnp.where(key == mk, jnp.int32(0x7FFFFFFF), key)
    idx_ref[0] = jnp.concatenate(cols, axis=1)


def _knn_and_project(x_pad, xt_pad, wct):
    b, n, _ = x_pad.shape
    m = xt_pad.shape[2]
    qt = 256
    grid = (b, n // qt)
    return pl.pallas_call(
        functools.partial(_knn_proj_body, m=m, k=K, r2=RADIUS * RADIUS),
        grid=grid,
        in_specs=[
            pl.BlockSpec((1, qt, 8), lambda i, j: (i, j, 0)),
            pl.BlockSpec((1, 8, m), lambda i, j: (i, 0, 0)),
            pl.BlockSpec((8, C), lambda i, j: (0, 0)),
        ],
        out_specs=[
            pl.BlockSpec((1, qt, K), lambda i, j: (i, j, 0)),
            pl.BlockSpec((1, qt, C), lambda i, j: (i, j, 0)),
        ],
        out_shape=[
            jax.ShapeDtypeStruct((b, n, K), jnp.int32),
            jax.ShapeDtypeStruct((b, n, C), jnp.float32),
        ],
    )(x_pad, xt_pad, wct)


def _make_sc_gather_max(nq, qpw):
    info = plsc.get_sparse_core_info()
    nc = info.num_cores
    mesh = plsc.VectorSubcoreMesh(core_axis_name="c", subcore_axis_name="s")
    inv_r = 1.0 / RADIUS

    nbuf = 4

    @functools.partial(
        pl.kernel,
        mesh=mesh,
        compiler_params=pltpu.CompilerParams(use_tc_tiling_on_sc=False),
        out_type=jax.ShapeDtypeStruct((nq, NOUT), jnp.float32),
        scratch_types=[
            pltpu.VMEM((qpw * K,), jnp.int32),
            pltpu.VMEM((qpw, C), jnp.float32),
            pltpu.VMEM((nbuf, K, C), jnp.float32),
            pltpu.VMEM((qpw, NOUT), jnp.float32),
            pltpu.VMEM((NOUT,), jnp.float32),
        ] + [pltpu.SemaphoreType.DMA] * nbuf,
    )
    def sc_kernel(p_hbm, idx_hbm, b0_hbm, out_hbm,
                  idx_v, pq_v, rows_v, out_v, b0_v, *sems):
        wid = lax.axis_index("s") * nc + lax.axis_index("c")
        base = wid * qpw
        pltpu.sync_copy(idx_hbm.at[pl.ds(base * K, qpw * K)], idx_v)
        pltpu.sync_copy(p_hbm.at[pl.ds(base, qpw)], pq_v)
        pltpu.sync_copy(b0_hbm, b0_v)
        b0v = b0_v[...]

        def dma(q, j):
            return pltpu.make_async_copy(
                p_hbm.at[idx_v.at[pl.ds(q * K, K)]], rows_v.at[j], sems[j])

        def compute(q, j):
            acc = NZERO * jnp.maximum(b0v, 0.0)
            for i in range(NGRP):
                g = rows_v[j, 0, pl.ds(i * 16, 16)]
                for r in range(1, K):
                    g = jnp.maximum(g, rows_v[j, r, pl.ds(i * 16, 16)])
                pq = pq_v[q, pl.ds(i * 16, 16)]
                t = jnp.maximum((g - pq) * inv_r + b0v, 0.0)
                acc = acc + (t + t if GRP_W[i] == 2.0 else t)
            out_v[q, :] = acc * (1.0 / NROT)

        for j in range(nbuf - 1):
            dma(j, j).start()

        def group(g8, carry):
            q0 = g8 * nbuf
            for j in range(nbuf):
                q = q0 + j
                nxt = q + nbuf - 1

                @pl.when(nxt < qpw)
                def _():
                    dma(nxt, (j + nbuf - 1) % nbuf).start()

                dma(q, j).wait()
                compute(q, j)
            return carry

        lax.fori_loop(0, qpw // nbuf, group, 0)
        pltpu.sync_copy(out_v, out_hbm.at[pl.ds(base, qpw)])

    return sc_kernel


def kernel(x, W0, b0, Rs):
    b, n, _ = x.shape
    d = jnp.einsum('ijj->ij', Rs)
    nnz = jnp.sum(jnp.abs(d) > 0.5, axis=1)
    d_sorted = d[jnp.argsort(-nnz, stable=True)]
    d_full = d_sorted[:4]
    d_ax12 = d_sorted[4:16]
    akey = jnp.argmax(jnp.abs(d_ax12), axis=1) * 2 + (
        jnp.sum(d_ax12, axis=1) > 0).astype(jnp.int32)
    d_axis = d_ax12[jnp.argsort(akey, stable=True)][::2]
    dg = jnp.concatenate([d_full, d_axis], axis=0)
    wc = dg[:, None, :] * W0[None, :, :]
    wct = jnp.swapaxes(wc.reshape(C, 3), 0, 1)
    wct = jnp.pad(wct, ((0, 5), (0, 0)))
    x_pad = jnp.pad(x, ((0, 0), (0, 0), (0, 5)))
    xt_pad = jnp.pad(jnp.swapaxes(x, 1, 2), ((0, 0), (0, 5), (0, 0)))

    idx, p = _knn_and_project(x_pad, xt_pad, wct)
    nq = b * n
    nw = 32
    sc = _make_sc_gather_max(nq, nq // nw)
    out = sc(p.reshape(nq, C), idx.reshape(nq * K), b0)
    return jnp.swapaxes(out.reshape(b, n, NOUT), 1, 2)


# speedup vs baseline: 12.5574x; 1.3228x over previous
import functools

import jax
import jax.numpy as jnp
import numpy as np
from jax import lax
from jax.experimental import pallas as pl
from jax.experimental.pallas import tpu as pltpu
from jax.experimental.pallas import tpu_sc as plsc

K = 32
RADIUS = 0.15
NROT = 24
NOUT = 16
NGRP = 10
GRP_W = (1.0,) * 4 + (2.0,) * 6
NZERO = 8.0
C = NGRP * NOUT


def _knn_proj_body(xq_ref, xt_ref, wct_ref, idx_ref, p_ref, *, m, k, r2):
    b = pl.program_id(0)
    xq = xq_ref[0]
    xt = xt_ref[0]
    p_ref[0] = jnp.dot(xq, wct_ref[...], preferred_element_type=jnp.float32)
    x2 = jnp.sum(xq * xq, axis=1, keepdims=True)
    y2 = jnp.sum(xt * xt, axis=0, keepdims=True)
    d2 = x2 + y2 - 2.0 * jnp.dot(xq, xt, preferred_element_type=jnp.float32)
    d2 = jnp.maximum(d2, 0.0)
    iota = lax.broadcasted_iota(jnp.int32, d2.shape, 1)
    keyb = (lax.bitcast_convert_type(jnp.maximum(d2, 1e-20), jnp.int32)
            & jnp.int32(~0x7FF)) | iota
    key = lax.bitcast_convert_type(keyb, jnp.float32)
    r2_key = jnp.float32(np.int32(
        (np.float32(r2).view(np.int32) & ~0x7FF) | 0x7FF
    ).view(np.float32))
    offset = b * m
    first = None
    cols = []
    for _ in range(k):
        mk = jnp.min(key, axis=1, keepdims=True)
        am = lax.bitcast_convert_type(mk, jnp.int32) & jnp.int32(0x7FF)
        if first is None:
            first = am
        cols.append(jnp.where(mk <= r2_key, am, first) + offset)
        key = jnp.where(key == mk, jnp.float32(jnp.inf), key)
    idx_ref[0] = jnp.concatenate(cols, axis=1)


def _knn_and_project(x_pad, xt_pad, wct):
    b, n, _ = x_pad.shape
    m = xt_pad.shape[2]
    qt = 256
    grid = (b, n // qt)
    return pl.pallas_call(
        functools.partial(_knn_proj_body, m=m, k=K, r2=RADIUS * RADIUS),
        grid=grid,
        in_specs=[
            pl.BlockSpec((1, qt, 8), lambda i, j: (i, j, 0)),
            pl.BlockSpec((1, 8, m), lambda i, j: (i, 0, 0)),
            pl.BlockSpec((8, C), lambda i, j: (0, 0)),
        ],
        out_specs=[
            pl.BlockSpec((1, qt, K), lambda i, j: (i, j, 0)),
            pl.BlockSpec((1, qt, C), lambda i, j: (i, j, 0)),
        ],
        out_shape=[
            jax.ShapeDtypeStruct((b, n, K), jnp.int32),
            jax.ShapeDtypeStruct((b, n, C), jnp.float32),
        ],
    )(x_pad, xt_pad, wct)


def _make_sc_gather_max(nq, qpw):
    info = plsc.get_sparse_core_info()
    nc = info.num_cores
    mesh = plsc.VectorSubcoreMesh(core_axis_name="c", subcore_axis_name="s")
    inv_r = 1.0 / RADIUS

    nbuf = 4

    @functools.partial(
        pl.kernel,
        mesh=mesh,
        compiler_params=pltpu.CompilerParams(use_tc_tiling_on_sc=False),
        out_type=jax.ShapeDtypeStruct((nq, NOUT), jnp.float32),
        scratch_types=[
            pltpu.VMEM((qpw * K,), jnp.int32),
            pltpu.VMEM((qpw, C), jnp.float32),
            pltpu.VMEM((nbuf, K, C), jnp.float32),
            pltpu.VMEM((qpw, NOUT), jnp.float32),
            pltpu.VMEM((NOUT,), jnp.float32),
        ] + [pltpu.SemaphoreType.DMA] * nbuf,
    )
    def sc_kernel(p_hbm, idx_hbm, b0_hbm, out_hbm,
                  idx_v, pq_v, rows_v, out_v, b0_v, *sems):
        wid = lax.axis_index("s") * nc + lax.axis_index("c")
        base = wid * qpw
        pltpu.sync_copy(idx_hbm.at[pl.ds(base * K, qpw * K)], idx_v)
        pltpu.sync_copy(p_hbm.at[pl.ds(base, qpw)], pq_v)
        pltpu.sync_copy(b0_hbm, b0_v)
        b0v = b0_v[...]

        def dma(q, j):
            return pltpu.make_async_copy(
                p_hbm.at[idx_v.at[pl.ds(q * K, K)]], rows_v.at[j], sems[j])

        def compute(q, j):
            acc = NZERO * jnp.maximum(b0v, 0.0)
            for i in range(NGRP):
                g = rows_v[j, 0, pl.ds(i * 16, 16)]
                for r in range(1, K):
                    g = jnp.maximum(g, rows_v[j, r, pl.ds(i * 16, 16)])
                pq = pq_v[q, pl.ds(i * 16, 16)]
                t = jnp.maximum((g - pq) * inv_r + b0v, 0.0)
                acc = acc + (t + t if GRP_W[i] == 2.0 else t)
            out_v[q, :] = acc * (1.0 / NROT)

        for j in range(nbuf - 1):
            dma(j, j).start()

        def group(g8, carry):
            q0 = g8 * nbuf
            for j in range(nbuf):
                q = q0 + j
                nxt = q + nbuf - 1

                @pl.when(nxt < qpw)
                def _():
                    dma(nxt, (j + nbuf - 1) % nbuf).start()

                dma(q, j).wait()
                compute(q, j)
            return carry

        lax.fori_loop(0, qpw // nbuf, group, 0)
        pltpu.sync_copy(out_v, out_hbm.at[pl.ds(base, qpw)])

    return sc_kernel


def kernel(x, W0, b0, Rs):
    b, n, _ = x.shape
    d = jnp.einsum('ijj->ij', Rs)
    nnz = jnp.sum(jnp.abs(d) > 0.5, axis=1)
    d_sorted = d[jnp.argsort(-nnz, stable=True)]
    d_full = d_sorted[:4]
    d_ax12 = d_sorted[4:16]
    akey = jnp.argmax(jnp.abs(d_ax12), axis=1) * 2 + (
        jnp.sum(d_ax12, axis=1) > 0).astype(jnp.int32)
    d_axis = d_ax12[jnp.argsort(akey, stable=True)][::2]
    dg = jnp.concatenate([d_full, d_axis], axis=0)
    wc = dg[:, None, :] * W0[None, :, :]
    wct = jnp.swapaxes(wc.reshape(C, 3), 0, 1)
    wct = jnp.pad(wct, ((0, 5), (0, 0)))
    x_pad = jnp.pad(x, ((0, 0), (0, 0), (0, 5)))
    xt_pad = jnp.pad(jnp.swapaxes(x, 1, 2), ((0, 0), (0, 5), (0, 0)))

    idx, p = _knn_and_project(x_pad, xt_pad, wct)
    nq = b * n
    nw = 32
    sc = _make_sc_gather_max(nq, nq // nw)
    out = sc(p.reshape(nq, C), idx.reshape(nq * K), b0)
    return jnp.swapaxes(out.reshape(b, n, NOUT), 1, 2)


# speedup vs baseline: 14.0164x; 1.1162x over previous
import functools

import jax
import jax.numpy as jnp
import numpy as np
from jax import lax
from jax.experimental import pallas as pl
from jax.experimental.pallas import tpu as pltpu
from jax.experimental.pallas import tpu_sc as plsc

K = 32
RADIUS = 0.15
NROT = 24
NOUT = 16
NGRP = 10
GRP_W = (1.0,) * 4 + (2.0,) * 6
NZERO = 8.0
C = NGRP * NOUT


def _proj_body(xq_ref, wct_ref, p_ref):
    p_ref[0] = jnp.dot(xq_ref[0], wct_ref[...],
                       preferred_element_type=jnp.float32)


def _project(x_pad, wct):
    b, n, _ = x_pad.shape
    qt = 512
    return pl.pallas_call(
        _proj_body,
        grid=(b, n // qt),
        in_specs=[
            pl.BlockSpec((1, qt, 8), lambda i, j: (i, j, 0)),
            pl.BlockSpec((8, C), lambda i, j: (0, 0)),
        ],
        out_specs=pl.BlockSpec((1, qt, C), lambda i, j: (i, j, 0)),
        out_shape=jax.ShapeDtypeStruct((b, n, C), jnp.float32),
    )(x_pad, wct)


def _knn_body(xq_ref, xt_ref, idx_ref, *, m, k, r2):
    b = pl.program_id(0)
    xq = xq_ref[0]
    xt = xt_ref[0]
    x2 = jnp.sum(xq * xq, axis=1, keepdims=True)
    y2 = jnp.sum(xt * xt, axis=0, keepdims=True)
    d2 = x2 + y2 - 2.0 * jnp.dot(xq, xt, preferred_element_type=jnp.float32)
    d2 = jnp.maximum(d2, 0.0)
    iota = lax.broadcasted_iota(jnp.int32, d2.shape, 1)
    keyb = (lax.bitcast_convert_type(jnp.maximum(d2, 1e-20), jnp.int32)
            & jnp.int32(~0x7FF)) | iota
    key = lax.bitcast_convert_type(keyb, jnp.float32)
    r2_key = jnp.float32(np.int32(
        (np.float32(r2).view(np.int32) & ~0x7FF) | 0x7FF
    ).view(np.float32))
    offset = b * m
    first = None
    cols = []
    for _ in range(k):
        mk = jnp.min(key, axis=1, keepdims=True)
        am = lax.bitcast_convert_type(mk, jnp.int32) & jnp.int32(0x7FF)
        if first is None:
            first = am
        cols.append(jnp.where(mk <= r2_key, am, first) + offset)
        key = jnp.where(key == mk, jnp.float32(jnp.inf), key)
    idx_ref[0] = jnp.concatenate(cols, axis=1)


def _knn(xq_pad, xt_pad):
    b, cq, _ = xq_pad.shape
    m = xt_pad.shape[2]
    qt = 256
    grid = (b, cq // qt)
    return pl.pallas_call(
        functools.partial(_knn_body, m=m, k=K, r2=RADIUS * RADIUS),
        grid=grid,
        in_specs=[
            pl.BlockSpec((1, qt, 8), lambda i, j: (i, j, 0)),
            pl.BlockSpec((1, 8, m), lambda i, j: (i, 0, 0)),
        ],
        out_specs=pl.BlockSpec((1, qt, K), lambda i, j: (i, j, 0)),
        out_shape=jax.ShapeDtypeStruct((b, cq, K), jnp.int32),
    )(xq_pad, xt_pad)


def _make_sc_gather_max(nq, qpw):
    info = plsc.get_sparse_core_info()
    nc = info.num_cores
    mesh = plsc.VectorSubcoreMesh(core_axis_name="c", subcore_axis_name="s")
    inv_r = 1.0 / RADIUS

    nbuf = 4

    @functools.partial(
        pl.kernel,
        mesh=mesh,
        compiler_params=pltpu.CompilerParams(use_tc_tiling_on_sc=False),
        out_type=jax.ShapeDtypeStruct((nq, NOUT), jnp.float32),
        scratch_types=[
            pltpu.VMEM((qpw * K,), jnp.int32),
            pltpu.VMEM((qpw, C), jnp.float32),
            pltpu.VMEM((nbuf, K, C), jnp.float32),
            pltpu.VMEM((qpw, NOUT), jnp.float32),
            pltpu.VMEM((NOUT,), jnp.float32),
        ] + [pltpu.SemaphoreType.DMA] * nbuf,
    )
    def sc_kernel(p_hbm, pq_hbm, idx_hbm, b0_hbm, out_hbm,
                  idx_v, pq_v, rows_v, out_v, b0_v, *sems):
        wid = lax.axis_index("s") * nc + lax.axis_index("c")
        base = wid * qpw
        pltpu.sync_copy(idx_hbm.at[pl.ds(base * K, qpw * K)], idx_v)
        pltpu.sync_copy(pq_hbm.at[pl.ds(base, qpw)], pq_v)
        pltpu.sync_copy(b0_hbm, b0_v)
        b0v = b0_v[...]

        def dma(q, j):
            return pltpu.make_async_copy(
                p_hbm.at[idx_v.at[pl.ds(q * K, K)]], rows_v.at[j], sems[j])

        def compute(q, j):
            acc = NZERO * jnp.maximum(b0v, 0.0)
            for i in range(NGRP):
                g = rows_v[j, 0, pl.ds(i * 16, 16)]
                for r in range(1, K):
                    g = jnp.maximum(g, rows_v[j, r, pl.ds(i * 16, 16)])
                pq = pq_v[q, pl.ds(i * 16, 16)]
                t = jnp.maximum((g - pq) * inv_r + b0v, 0.0)
                acc = acc + (t + t if GRP_W[i] == 2.0 else t)
            out_v[q, :] = acc * (1.0 / NROT)

        for j in range(nbuf - 1):
            dma(j, j).start()

        def group(g8, carry):
            q0 = g8 * nbuf
            for j in range(nbuf):
                q = q0 + j
                nxt = q + nbuf - 1

                @pl.when(nxt < qpw)
                def _():
                    dma(nxt, (j + nbuf - 1) % nbuf).start()

                dma(q, j).wait()
                compute(q, j)
            return carry

        lax.fori_loop(0, qpw // nbuf, group, 0)
        pltpu.sync_copy(out_v, out_hbm.at[pl.ds(base, qpw)])

    return sc_kernel


def kernel(x, W0, b0, Rs):
    b, n, _ = x.shape
    d = jnp.einsum('ijj->ij', Rs)
    nnz = jnp.sum(jnp.abs(d) > 0.5, axis=1)
    d_sorted = d[jnp.argsort(-nnz, stable=True)]
    d_full = d_sorted[:4]
    d_ax12 = d_sorted[4:16]
    akey = jnp.argmax(jnp.abs(d_ax12), axis=1) * 2 + (
        jnp.sum(d_ax12, axis=1) > 0).astype(jnp.int32)
    d_axis = d_ax12[jnp.argsort(akey, stable=True)][::2]
    dg = jnp.concatenate([d_full, d_axis], axis=0)
    wc = dg[:, None, :] * W0[None, :, :]
    wct = jnp.swapaxes(wc.reshape(C, 3), 0, 1)
    wct = jnp.pad(wct, ((0, 5), (0, 0)))
    x_pad = jnp.pad(x, ((0, 0), (0, 0), (0, 5)))
    xt_pad = jnp.pad(jnp.swapaxes(x, 1, 2), ((0, 0), (0, 5), (0, 0)))

    p = _project(x_pad, wct)
    pf = p.reshape(b * n, C)
    nch = 2
    cq = n // nch
    nw = 32
    sc = _make_sc_gather_max(b * cq, b * cq // nw)
    outs = []
    for h in range(nch):
        xq_h = x_pad[:, h * cq:(h + 1) * cq]
        idx_h = _knn(xq_h, xt_pad)
        pq_h = p[:, h * cq:(h + 1) * cq].reshape(b * cq, C)
        out_h = sc(pf, pq_h, idx_h.reshape(b * cq * K), b0)
        outs.append(out_h.reshape(b, cq, NOUT))
    out = jnp.concatenate(outs, axis=1)
    return jnp.swapaxes(out, 1, 2)


# speedup vs baseline: 14.0620x; 1.0033x over previous
import functools

import jax
import jax.numpy as jnp
import numpy as np
from jax import lax
from jax.experimental import pallas as pl
from jax.experimental.pallas import tpu as pltpu
from jax.experimental.pallas import tpu_sc as plsc

K = 32
RADIUS = 0.15
NROT = 24
NOUT = 16
NGRP = 10
GRP_W = (1.0,) * 4 + (2.0,) * 6
NZERO = 8.0
C = NGRP * NOUT


def _proj_body(xq_ref, wct_ref, p_ref):
    p_ref[0] = jnp.dot(xq_ref[0], wct_ref[...],
                       preferred_element_type=jnp.float32)


def _project(x_pad, wct):
    b, n, _ = x_pad.shape
    qt = 512
    return pl.pallas_call(
        _proj_body,
        grid=(b, n // qt),
        in_specs=[
            pl.BlockSpec((1, qt, 8), lambda i, j: (i, j, 0)),
            pl.BlockSpec((8, C), lambda i, j: (0, 0)),
        ],
        out_specs=pl.BlockSpec((1, qt, C), lambda i, j: (i, j, 0)),
        out_shape=jax.ShapeDtypeStruct((b, n, C), jnp.float32),
    )(x_pad, wct)


def _knn_body(xq_ref, xt_ref, idx_ref, *, m, k, r2):
    b = pl.program_id(0)
    xq = xq_ref[0]
    xt = xt_ref[0]
    x2 = jnp.sum(xq * xq, axis=1, keepdims=True)
    y2 = jnp.sum(xt * xt, axis=0, keepdims=True)
    d2 = x2 + y2 - 2.0 * jnp.dot(xq, xt, preferred_element_type=jnp.float32)
    d2 = jnp.maximum(d2, 0.0)
    iota = lax.broadcasted_iota(jnp.int32, d2.shape, 1)
    keyb = (lax.bitcast_convert_type(jnp.maximum(d2, 1e-20), jnp.int32)
            & jnp.int32(~0x7FF)) | iota
    key = lax.bitcast_convert_type(keyb, jnp.float32)
    r2_key = jnp.float32(np.int32(
        (np.float32(r2).view(np.int32) & ~0x7FF) | 0x7FF
    ).view(np.float32))
    offset = b * m
    q4 = m // 4
    q1, q2, q3, q4v = (key[:, i * q4:(i + 1) * q4] for i in range(4))
    a_lo, a_hi = jnp.minimum(q1, q2), jnp.maximum(q1, q2)
    b_lo, b_hi = jnp.minimum(q3, q4v), jnp.maximum(q3, q4v)
    l1, c_hi = jnp.minimum(a_lo, b_lo), jnp.maximum(a_lo, b_lo)
    c_lo, l4 = jnp.minimum(a_hi, b_hi), jnp.maximum(a_hi, b_hi)
    l2, l3 = jnp.minimum(c_hi, c_lo), jnp.maximum(c_hi, c_lo)
    inf = jnp.float32(jnp.inf)
    first = None
    cols = []
    for _ in range(k):
        mk = jnp.min(l1, axis=1, keepdims=True)
        am = lax.bitcast_convert_type(mk, jnp.int32) & jnp.int32(0x7FF)
        if first is None:
            first = am
        cols.append(jnp.where(mk <= r2_key, am, first) + offset)
        e = l1 == mk
        l1 = jnp.where(e, l2, l1)
        l2 = jnp.where(e, l3, l2)
        l3 = jnp.where(e, l4, l3)
        l4 = jnp.where(e, inf, l4)
    idx_ref[0] = jnp.concatenate(cols, axis=1)


def _knn(xq_pad, xt_pad):
    b, cq, _ = xq_pad.shape
    m = xt_pad.shape[2]
    qt = 256
    grid = (b, cq // qt)
    return pl.pallas_call(
        functools.partial(_knn_body, m=m, k=K, r2=RADIUS * RADIUS),
        grid=grid,
        in_specs=[
            pl.BlockSpec((1, qt, 8), lambda i, j: (i, j, 0)),
            pl.BlockSpec((1, 8, m), lambda i, j: (i, 0, 0)),
        ],
        out_specs=pl.BlockSpec((1, qt, K), lambda i, j: (i, j, 0)),
        out_shape=jax.ShapeDtypeStruct((b, cq, K), jnp.int32),
    )(xq_pad, xt_pad)


def _make_sc_gather_max(nq, qpw):
    info = plsc.get_sparse_core_info()
    nc = info.num_cores
    mesh = plsc.VectorSubcoreMesh(core_axis_name="c", subcore_axis_name="s")
    inv_r = 1.0 / RADIUS

    nbuf = 4

    @functools.partial(
        pl.kernel,
        mesh=mesh,
        compiler_params=pltpu.CompilerParams(use_tc_tiling_on_sc=False),
        out_type=jax.ShapeDtypeStruct((nq, NOUT), jnp.float32),
        scratch_types=[
            pltpu.VMEM((qpw * K,), jnp.int32),
            pltpu.VMEM((qpw, C), jnp.float32),
            pltpu.VMEM((nbuf, K, C), jnp.float32),
            pltpu.VMEM((qpw, NOUT), jnp.float32),
            pltpu.VMEM((NOUT,), jnp.float32),
        ] + [pltpu.SemaphoreType.DMA] * nbuf,
    )
    def sc_kernel(p_hbm, pq_hbm, idx_hbm, b0_hbm, out_hbm,
                  idx_v, pq_v, rows_v, out_v, b0_v, *sems):
        wid = lax.axis_index("s") * nc + lax.axis_index("c")
        base = wid * qpw
        pltpu.sync_copy(idx_hbm.at[pl.ds(base * K, qpw * K)], idx_v)
        pltpu.sync_copy(pq_hbm.at[pl.ds(base, qpw)], pq_v)
        pltpu.sync_copy(b0_hbm, b0_v)
        b0v = b0_v[...]

        def dma(q, j):
            return pltpu.make_async_copy(
                p_hbm.at[idx_v.at[pl.ds(q * K, K)]], rows_v.at[j], sems[j])

        def compute(q, j):
            acc = NZERO * jnp.maximum(b0v, 0.0)
            for i in range(NGRP):
                g = rows_v[j, 0, pl.ds(i * 16, 16)]
                for r in range(1, K):
                    g = jnp.maximum(g, rows_v[j, r, pl.ds(i * 16, 16)])
                pq = pq_v[q, pl.ds(i * 16, 16)]
                t = jnp.maximum((g - pq) * inv_r + b0v, 0.0)
                acc = acc + (t + t if GRP_W[i] == 2.0 else t)
            out_v[q, :] = acc * (1.0 / NROT)

        for j in range(nbuf - 1):
            dma(j, j).start()

        def group(g8, carry):
            q0 = g8 * nbuf
            for j in range(nbuf):
                q = q0 + j
                nxt = q + nbuf - 1

                @pl.when(nxt < qpw)
                def _():
                    dma(nxt, (j + nbuf - 1) % nbuf).start()

                dma(q, j).wait()
                compute(q, j)
            return carry

        lax.fori_loop(0, qpw // nbuf, group, 0)
        pltpu.sync_copy(out_v, out_hbm.at[pl.ds(base, qpw)])

    return sc_kernel


def kernel(x, W0, b0, Rs):
    b, n, _ = x.shape
    d = jnp.einsum('ijj->ij', Rs)
    nnz = jnp.sum(jnp.abs(d) > 0.5, axis=1)
    d_sorted = d[jnp.argsort(-nnz, stable=True)]
    d_full = d_sorted[:4]
    d_ax12 = d_sorted[4:16]
    akey = jnp.argmax(jnp.abs(d_ax12), axis=1) * 2 + (
        jnp.sum(d_ax12, axis=1) > 0).astype(jnp.int32)
    d_axis = d_ax12[jnp.argsort(akey, stable=True)][::2]
    dg = jnp.concatenate([d_full, d_axis], axis=0)
    wc = dg[:, None, :] * W0[None, :, :]
    wct = jnp.swapaxes(wc.reshape(C, 3), 0, 1)
    wct = jnp.pad(wct, ((0, 5), (0, 0)))
    x_pad = jnp.pad(x, ((0, 0), (0, 0), (0, 5)))
    xt_pad = jnp.pad(jnp.swapaxes(x, 1, 2), ((0, 0), (0, 5), (0, 0)))

    p = _project(x_pad, wct)
    pf = p.reshape(b * n, C)
    nch = 2
    cq = n // nch
    nw = 32
    sc = _make_sc_gather_max(b * cq, b * cq // nw)
    outs = []
    for h in range(nch):
        xq_h = x_pad[:, h * cq:(h + 1) * cq]
        idx_h = _knn(xq_h, xt_pad)
        pq_h = p[:, h * cq:(h + 1) * cq].reshape(b * cq, C)
        out_h = sc(pf, pq_h, idx_h.reshape(b * cq * K), b0)
        outs.append(out_h.reshape(b, cq, NOUT))
    out = jnp.concatenate(outs, axis=1)
    return jnp.swapaxes(out, 1, 2)


# speedup vs baseline: 15.0088x; 1.0673x over previous
import functools

import jax
import jax.numpy as jnp
import numpy as np
from jax import lax
from jax.experimental import pallas as pl
from jax.experimental.pallas import tpu as pltpu
from jax.experimental.pallas import tpu_sc as plsc

K = 32
RADIUS = 0.15
NROT = 24
NOUT = 16
NGRP = 10
GRP_W = (1.0,) * 4 + (2.0,) * 6
NZERO = 8.0
C = NGRP * NOUT


def _proj_body(xq_ref, wct_ref, p_ref):
    p_ref[0] = jnp.dot(xq_ref[0], wct_ref[...],
                       preferred_element_type=jnp.float32)


def _project(x_pad, wct):
    b, n, _ = x_pad.shape
    qt = 512
    return pl.pallas_call(
        _proj_body,
        grid=(b, n // qt),
        in_specs=[
            pl.BlockSpec((1, qt, 8), lambda i, j: (i, j, 0)),
            pl.BlockSpec((8, C), lambda i, j: (0, 0)),
        ],
        out_specs=pl.BlockSpec((1, qt, C), lambda i, j: (i, j, 0)),
        out_shape=jax.ShapeDtypeStruct((b, n, C), jnp.float32),
    )(x_pad, wct)


def _knn_body(xq_ref, xt_ref, idx_ref, *, m, k, r2):
    b = pl.program_id(0)
    xq = xq_ref[0]
    xt = xt_ref[0]
    x2 = jnp.sum(xq * xq, axis=1, keepdims=True)
    y2 = jnp.sum(xt * xt, axis=0, keepdims=True)
    d2 = x2 + y2 - 2.0 * jnp.dot(xq, xt, preferred_element_type=jnp.float32)
    d2 = jnp.maximum(d2, 0.0)
    iota = lax.broadcasted_iota(jnp.int32, d2.shape, 1)
    keyb = (lax.bitcast_convert_type(jnp.maximum(d2, 1e-20), jnp.int32)
            & jnp.int32(~0x7FF)) | iota
    key = lax.bitcast_convert_type(keyb, jnp.float32)
    r2_key = jnp.float32(np.int32(
        (np.float32(r2).view(np.int32) & ~0x7FF) | 0x7FF
    ).view(np.float32))
    offset = b * m
    q4 = m // 4
    q1, q2, q3, q4v = (key[:, i * q4:(i + 1) * q4] for i in range(4))
    a_lo, a_hi = jnp.minimum(q1, q2), jnp.maximum(q1, q2)
    b_lo, b_hi = jnp.minimum(q3, q4v), jnp.maximum(q3, q4v)
    l1, c_hi = jnp.minimum(a_lo, b_lo), jnp.maximum(a_lo, b_lo)
    c_lo, l4 = jnp.minimum(a_hi, b_hi), jnp.maximum(a_hi, b_hi)
    l2, l3 = jnp.minimum(c_hi, c_lo), jnp.maximum(c_hi, c_lo)
    inf = jnp.float32(jnp.inf)
    first = None
    cols = []
    for _ in range(k):
        mk = jnp.min(l1, axis=1, keepdims=True)
        am = lax.bitcast_convert_type(mk, jnp.int32) & jnp.int32(0x7FF)
        if first is None:
            first = am
        cols.append(jnp.where(mk <= r2_key, am, first) + offset)
        e = l1 == mk
        l1 = jnp.where(e, l2, l1)
        l2 = jnp.where(e, l3, l2)
        l3 = jnp.where(e, l4, l3)
        l4 = jnp.where(e, inf, l4)
    idx_ref[0] = jnp.concatenate(cols, axis=1)


def _knn(xq_pad, xt_pad):
    b, cq, _ = xq_pad.shape
    m = xt_pad.shape[2]
    qt = 256
    grid = (b, cq // qt)
    return pl.pallas_call(
        functools.partial(_knn_body, m=m, k=K, r2=RADIUS * RADIUS),
        grid=grid,
        in_specs=[
            pl.BlockSpec((1, qt, 8), lambda i, j: (i, j, 0)),
            pl.BlockSpec((1, 8, m), lambda i, j: (i, 0, 0)),
        ],
        out_specs=pl.BlockSpec((1, qt, K), lambda i, j: (i, j, 0)),
        out_shape=jax.ShapeDtypeStruct((b, cq, K), jnp.int32),
    )(xq_pad, xt_pad)


def _make_sc_gather_max(nq, qpw, seg_off, n, nb):
    info = plsc.get_sparse_core_info()
    nc = info.num_cores
    mesh = plsc.VectorSubcoreMesh(core_axis_name="c", subcore_axis_name="s")
    inv_r = 1.0 / RADIUS
    nw = nq // qpw
    wpb = nw // nb

    nbuf = 4

    @functools.partial(
        pl.kernel,
        mesh=mesh,
        compiler_params=pltpu.CompilerParams(use_tc_tiling_on_sc=False),
        out_type=jax.ShapeDtypeStruct((nq, NOUT), jnp.float32),
        scratch_types=[
            pltpu.VMEM((qpw * K,), jnp.int32),
            pltpu.VMEM((qpw, C), jnp.float32),
            pltpu.VMEM((nbuf, K, C), jnp.float32),
            pltpu.VMEM((qpw, NOUT), jnp.float32),
            pltpu.VMEM((NOUT,), jnp.float32),
        ] + [pltpu.SemaphoreType.DMA] * nbuf,
    )
    def sc_kernel(p_hbm, idx_hbm, b0_hbm, out_hbm,
                  idx_v, pq_v, rows_v, out_v, b0_v, *sems):
        wid = lax.axis_index("s") * nc + lax.axis_index("c")
        base = wid * qpw
        pq_base = (wid // wpb) * n + seg_off + (wid % wpb) * qpw
        pltpu.sync_copy(idx_hbm.at[pl.ds(base * K, qpw * K)], idx_v)
        pltpu.sync_copy(p_hbm.at[pl.ds(pq_base, qpw)], pq_v)
        pltpu.sync_copy(b0_hbm, b0_v)
        b0v = b0_v[...]

        def dma(q, j):
            return pltpu.make_async_copy(
                p_hbm.at[idx_v.at[pl.ds(q * K, K)]], rows_v.at[j], sems[j])

        def compute(q, j):
            acc = NZERO * jnp.maximum(b0v, 0.0)
            for i in range(NGRP):
                g = rows_v[j, 0, pl.ds(i * 16, 16)]
                for r in range(1, K):
                    g = jnp.maximum(g, rows_v[j, r, pl.ds(i * 16, 16)])
                pq = pq_v[q, pl.ds(i * 16, 16)]
                t = jnp.maximum((g - pq) * inv_r + b0v, 0.0)
                acc = acc + (t + t if GRP_W[i] == 2.0 else t)
            out_v[q, :] = acc * (1.0 / NROT)

        for j in range(nbuf - 1):
            dma(j, j).start()

        def group(g8, carry):
            q0 = g8 * nbuf
            for j in range(nbuf):
                q = q0 + j
                nxt = q + nbuf - 1

                @pl.when(nxt < qpw)
                def _():
                    dma(nxt, (j + nbuf - 1) % nbuf).start()

                dma(q, j).wait()
                compute(q, j)
            return carry

        lax.fori_loop(0, qpw // nbuf, group, 0)
        pltpu.sync_copy(out_v, out_hbm.at[pl.ds(base, qpw)])

    return sc_kernel


def kernel(x, W0, b0, Rs):
    b, n, _ = x.shape
    d = jnp.einsum('ijj->ij', Rs)
    nnz = jnp.sum(jnp.abs(d) > 0.5, axis=1)
    d_sorted = d[jnp.argsort(-nnz, stable=True)]
    d_full = d_sorted[:4]
    d_ax12 = d_sorted[4:16]
    akey = jnp.argmax(jnp.abs(d_ax12), axis=1) * 2 + (
        jnp.sum(d_ax12, axis=1) > 0).astype(jnp.int32)
    d_axis = d_ax12[jnp.argsort(akey, stable=True)][::2]
    dg = jnp.concatenate([d_full, d_axis], axis=0)
    wc = dg[:, None, :] * W0[None, :, :]
    wct = jnp.swapaxes(wc.reshape(C, 3), 0, 1)
    wct = jnp.pad(wct, ((0, 5), (0, 0)))
    x_pad = jnp.pad(x, ((0, 0), (0, 0), (0, 5)))
    xt_pad = jnp.pad(jnp.swapaxes(x, 1, 2), ((0, 0), (0, 5), (0, 0)))

    p = _project(x_pad, wct)
    pf = p.reshape(b * n, C)
    nch = 4
    cq = n // nch
    nw = 32
    outs = []
    for h in range(nch):
        xq_h = x_pad[:, h * cq:(h + 1) * cq]
        idx_h = _knn(xq_h, xt_pad)
        sc = _make_sc_gather_max(b * cq, b * cq // nw, h * cq, n, b)
        out_h = sc(pf, idx_h.reshape(b * cq * K), b0)
        outs.append(out_h.reshape(b, cq, NOUT))
    out = jnp.concatenate(outs, axis=1)
    return jnp.swapaxes(out, 1, 2)
